# baseline (speedup 1.0000x reference)
import functools

import jax
import jax.numpy as jnp
from jax import lax
from jax.experimental import pallas as pl
from jax.experimental.pallas import tpu as pltpu
from jax.experimental.pallas import tpu_sc as plsc

N = 10000
E = 320000
D = 128
H = 128
K = 3
C = 2

NC = 2
NS = 16
NW = NC * NS
L = 16
CH = 128
EP = 327680
NCH2 = EP // CH
CPT = NCH2 // NW
NPAD = 10240
SLICE = NPAD // NS
BLK = 256
NBLK = NPAD // BLK

_mesh = plsc.VectorSubcoreMesh(core_axis_name="c", subcore_axis_name="s")
_sc_params = pltpu.CompilerParams(needs_layout_passes=False)


def _wid():
    return lax.axis_index("s") * NC + lax.axis_index("c")


@functools.partial(
    pl.kernel,
    out_type=jax.ShapeDtypeStruct((NC * 2 * NPAD,), jnp.float32),
    mesh=_mesh,
    compiler_params=_sc_params,
    scratch_types=[
        pltpu.VMEM((CH,), jnp.float32),
        pltpu.VMEM((5, CH), jnp.int32),
        pltpu.VMEM((5, CH), jnp.int32),
        pltpu.VMEM((5, CH), jnp.int32),
        pltpu.VMEM((5, CH), jnp.int32),
        pltpu.SemaphoreType.DMA,
        pltpu.SemaphoreType.DMA,
        pltpu.SemaphoreType.DMA,
        pltpu.SemaphoreType.DMA,
        pltpu.SemaphoreType.DMA,
        pltpu.SemaphoreType.DMA,
        pltpu.MemorySpace.VMEM_SHARED((NPAD,), jnp.float32),
        pltpu.MemorySpace.VMEM_SHARED((NPAD,), jnp.float32),
    ],
)
def _sc_deg(e5_h, ones_h, zeros1_h, out_h, ones_v, e0, e1, e2b, e3,
            is0, is1, is2, is3, os0, os1, dego_sh, degi_sh):
    c = lax.axis_index("c")
    s = lax.axis_index("s")
    wid = _wid()
    start = s * SLICE
    ebufs = (e0, e1, e2b, e3)
    isems = (is0, is1, is2, is3)
    osems = (os0, os1)

    def issue_in(t, b):
        pltpu.async_copy(e5_h.at[wid + NW * t], ebufs[b], isems[b])

    def wait_in(b):
        pltpu.make_async_copy(e5_h.at[0], ebufs[b], isems[b]).wait()

    def issue_out(b, p):
        pltpu.async_copy(ones_v, dego_sh.at[ebufs[b].at[0]], osems[p],
                         add=True)
        pltpu.async_copy(ones_v, degi_sh.at[ebufs[b].at[1]], osems[p],
                         add=True)

    def wait_out(b, p):
        pltpu.make_async_copy(ones_v, dego_sh.at[ebufs[b].at[0]],
                              osems[p]).wait()
        pltpu.make_async_copy(ones_v, degi_sh.at[ebufs[b].at[1]],
                              osems[p]).wait()

    pltpu.sync_copy(zeros1_h, dego_sh.at[pl.ds(start, SLICE)])
    pltpu.sync_copy(zeros1_h, degi_sh.at[pl.ds(start, SLICE)])
    pltpu.sync_copy(ones_h, ones_v)
    issue_in(0, 0)
    issue_in(1, 1)
    plsc.subcore_barrier()

    def outer(i, carry):
        t0 = i * 4
        for b in range(4):
            t = t0 + b
            wait_in(b)

            @pl.when(t >= 2)
            def _(b=b):
                wait_out((b + 2) % 4, b % 2)

            @pl.when(t + 2 < CPT)
            def _(t=t, b=b):
                issue_in(t + 2, (b + 2) % 4)

            issue_out(b, b % 2)
        return carry

    lax.fori_loop(0, CPT // 4, outer, 0)
    wait_out(2, 0)
    wait_out(3, 1)
    plsc.subcore_barrier()
    pltpu.sync_copy(dego_sh.at[pl.ds(start, SLICE)],
                    out_h.at[pl.ds((c * 2 + 0) * NPAD + start, SLICE)])
    pltpu.sync_copy(degi_sh.at[pl.ds(start, SLICE)],
                    out_h.at[pl.ds((c * 2 + 1) * NPAD + start, SLICE)])


BLK2 = 1024


def _tc2_body(deg, x, xn_out, ndcol_out):
    d = deg[...]
    ncol_s = lax.rsqrt(d[0] + d[2] + 1.0)
    ncol_d = lax.rsqrt(d[1] + d[3] + 1.0)
    xn_out[...] = x[...] * ncol_s
    ndcol_out[...] = ncol_d


def _tc2(dcols, x_pad):
    col = pl.BlockSpec((BLK2, 1), lambda i: (i, 0))
    mat = pl.BlockSpec((BLK2, D), lambda i: (i, 0))
    return pl.pallas_call(
        _tc2_body,
        grid=(NPAD // BLK2,),
        in_specs=[pl.BlockSpec((4, BLK2, 1), lambda i: (0, i, 0)), mat],
        out_specs=[mat, col],
        out_shape=[
            jax.ShapeDtypeStruct((NPAD, D), jnp.float32),
            jax.ShapeDtypeStruct((NPAD, 1), jnp.float32),
        ],
    )(dcols, x_pad)


@functools.partial(
    pl.kernel,
    out_type=jax.ShapeDtypeStruct((NC, NPAD, D), jnp.float32),
    mesh=_mesh,
    compiler_params=_sc_params,
    scratch_types=(
        [pltpu.VMEM((5, CH), jnp.int32)] * 8 +
        [pltpu.VMEM((CH, D), jnp.float32)] * 2 +
        [pltpu.SemaphoreType.DMA] * 8 +
        [pltpu.SemaphoreType.DMA] * 2 +
        [pltpu.SemaphoreType.DMA] * 2 +
        [pltpu.MemorySpace.VMEM_SHARED((NPAD, D), jnp.float32)]
    ),
)
def _sc_agg(xn_h, e5_h, zrows_h, out_h, *refs):
    ebufs = refs[0:8]
    rows = refs[8:10]
    isems = refs[10:18]
    gsems = refs[18:20]
    ssems = refs[20:22]
    acc_sh = refs[22]
    c = lax.axis_index("c")
    s = lax.axis_index("s")
    wid = _wid()
    start = s * SLICE

    def issue_in(t, b):
        pltpu.async_copy(e5_h.at[wid + NW * t], ebufs[b], isems[b])

    def wait_in(b):
        pltpu.make_async_copy(e5_h.at[0], ebufs[b], isems[b]).wait()

    def issue_gather(b, p):
        pltpu.async_copy(xn_h.at[ebufs[b].at[0]], rows[p], gsems[p])

    def wait_gather(b, p):
        pltpu.make_async_copy(xn_h.at[ebufs[b].at[0]], rows[p],
                              gsems[p]).wait()

    def issue_scatter(b, p):
        pltpu.async_copy(rows[p], acc_sh.at[ebufs[b].at[1]], ssems[p],
                         add=True)

    def wait_scatter(b, p):
        pltpu.make_async_copy(rows[p], acc_sh.at[ebufs[b].at[1]],
                              ssems[p]).wait()

    pltpu.sync_copy(zrows_h, acc_sh.at[pl.ds(start, SLICE)])
    for b in range(5):
        issue_in(b, b)
    plsc.subcore_barrier()

    def outer(i, carry):
        t0 = i * 8
        for b in range(8):
            t = t0 + b
            wait_in(b)

            @pl.when(t >= 2)
            def _(b=b):
                wait_scatter((b + 6) % 8, b % 2)

            @pl.when(t + 5 < CPT)
            def _(t=t, b=b):
                issue_in(t + 5, (b + 5) % 8)

            issue_gather(b, b % 2)

            @pl.when(t >= 1)
            def _(b=b):
                wait_gather((b + 7) % 8, (b + 1) % 2)
                issue_scatter((b + 7) % 8, (b + 1) % 2)

        return carry

    lax.fori_loop(0, CPT // 8, outer, 0)
    wait_gather(7, 1)
    issue_scatter(7, 1)
    wait_scatter(6, 0)
    wait_scatter(7, 1)
    plsc.subcore_barrier()
    pltpu.sync_copy(acc_sh.at[pl.ds(start, SLICE)],
                    out_h.at[c, pl.ds(start, SLICE)])


BLK4 = 512
NBLK4 = NPAD // BLK4


def _tc4_body(aggparts, xn, ndcol, conv_w, conv_b, W_w, W_b, aw2, ab,
              z_out, u_out, v_out, zsum_out, m16_out, b16_out, muv_acc):
    i = pl.program_id(0)
    A = (aggparts[0] + aggparts[1] + xn[...]) * ndcol[...]
    h = jnp.dot(A, conv_w[...], preferred_element_type=jnp.float32) + conv_b[...]
    Z = jnp.dot(h, W_w[...], preferred_element_type=jnp.float32) + W_b[...]
    UV = jnp.dot(Z, aw2[...], preferred_element_type=jnp.float32)
    z_out[...] = Z
    u_out[...] = UV[:, 0:1]
    v_out[...] = UV[:, 1:2]
    rows = lax.broadcasted_iota(jnp.int32, (BLK4, 1), 0) + i * BLK4
    valid = rows < N
    zs = jnp.sum(jnp.where(valid, Z, 0.0), axis=0, keepdims=True)
    mu = jnp.max(jnp.where(valid, UV, -3e38), axis=0, keepdims=True)

    @pl.when(i == 0)
    def _():
        zsum_out[...] = zs
        muv_acc[...] = mu

    @pl.when(i > 0)
    def _():
        zsum_out[...] = zsum_out[...] + zs
        muv_acc[...] = jnp.maximum(muv_acc[...], mu)

    @pl.when(i == NBLK4 - 1)
    def _():
        muv = muv_acc[...]
        m = jnp.maximum(muv[:, 0:1] + muv[:, 1:2] + ab[...], 0.0)
        m16_out[...] = jnp.broadcast_to(m, (1, L))
        b16_out[...] = jnp.broadcast_to(ab[...], (1, L))


def _tc4(aggparts, xn, ndcol, conv_w, conv_b, W_w, W_b, aw2, ab):
    return pl.pallas_call(
        _tc4_body,
        grid=(NBLK4,),
        in_specs=[
            pl.BlockSpec((NC, BLK4, D), lambda i: (0, i, 0)),
            pl.BlockSpec((BLK4, D), lambda i: (i, 0)),
            pl.BlockSpec((BLK4, 1), lambda i: (i, 0)),
            pl.BlockSpec((D, D), lambda i: (0, 0)),
            pl.BlockSpec((1, D), lambda i: (0, 0)),
            pl.BlockSpec((D, H), lambda i: (0, 0)),
            pl.BlockSpec((1, H), lambda i: (0, 0)),
            pl.BlockSpec((H, 2), lambda i: (0, 0)),
            pl.BlockSpec((1, 1), lambda i: (0, 0)),
        ],
        out_specs=[
            pl.BlockSpec((BLK4, H), lambda i: (i, 0)),
            pl.BlockSpec((BLK4, 1), lambda i: (i, 0)),
            pl.BlockSpec((BLK4, 1), lambda i: (i, 0)),
            pl.BlockSpec((1, H), lambda i: (0, 0)),
            pl.BlockSpec((1, L), lambda i: (0, 0)),
            pl.BlockSpec((1, L), lambda i: (0, 0)),
        ],
        out_shape=[
            jax.ShapeDtypeStruct((NPAD, H), jnp.float32),
            jax.ShapeDtypeStruct((NPAD, 1), jnp.float32),
            jax.ShapeDtypeStruct((NPAD, 1), jnp.float32),
            jax.ShapeDtypeStruct((1, H), jnp.float32),
            jax.ShapeDtypeStruct((1, L), jnp.float32),
            jax.ShapeDtypeStruct((1, L), jnp.float32),
        ],
        scratch_shapes=[pltpu.VMEM((1, 2), jnp.float32)],
    )(aggparts, xn, ndcol, conv_w, conv_b, W_w, W_b, aw2, ab)


@functools.partial(
    pl.kernel,
    out_type=jax.ShapeDtypeStruct((NC * K * NPAD,), jnp.float32),
    mesh=_mesh,
    compiler_params=_sc_params,
    scratch_types=[
        pltpu.VMEM((NPAD,), jnp.float32),
        pltpu.VMEM((NPAD,), jnp.float32),
        pltpu.VMEM((L,), jnp.float32),
        pltpu.VMEM((L,), jnp.float32),
        pltpu.VMEM((5, CH), jnp.int32),
        pltpu.VMEM((5, CH), jnp.int32),
        pltpu.VMEM((5, CH), jnp.int32),
        pltpu.VMEM((5, CH), jnp.int32),
        pltpu.VMEM((K, CH), jnp.float32),
        pltpu.VMEM((K, CH), jnp.float32),
        pltpu.SemaphoreType.DMA,
        pltpu.SemaphoreType.DMA,
        pltpu.SemaphoreType.DMA,
        pltpu.SemaphoreType.DMA,
        pltpu.SemaphoreType.DMA,
        pltpu.SemaphoreType.DMA,
        pltpu.MemorySpace.VMEM_SHARED((NPAD,), jnp.float32),
        pltpu.MemorySpace.VMEM_SHARED((NPAD,), jnp.float32),
        pltpu.MemorySpace.VMEM_SHARED((NPAD,), jnp.float32),
    ],
)
def _sc_den(u_h, v_h, m_h, b_h, e5_h, zeros1_h, out_h,
            u_v, v_v, m_v, b_v, e0, e1, e2b, e3, p0, p1,
            is0, is1, is2, is3, os0, os1, d0, d1, d2):
    c = lax.axis_index("c")
    s = lax.axis_index("s")
    wid = _wid()
    start = s * SLICE
    ebufs = (e0, e1, e2b, e3)
    pbufs = (p0, p1)
    isems = (is0, is1, is2, is3)
    osems = (os0, os1)
    dens = (d0, d1, d2)

    def issue_in(t, b):
        pltpu.async_copy(e5_h.at[wid + NW * t], ebufs[b], isems[b])

    def wait_in(b):
        pltpu.make_async_copy(e5_h.at[0], ebufs[b], isems[b]).wait()

    def issue_out(b, p):
        for k in range(K):
            pltpu.async_copy(pbufs[p].at[k], dens[k].at[ebufs[b].at[1]],
                             osems[p], add=True)

    def wait_out(b, p):
        for k in range(K):
            pltpu.make_async_copy(pbufs[p].at[k],
                                  dens[k].at[ebufs[b].at[1]],
                                  osems[p]).wait()

    for dsh in dens:
        pltpu.sync_copy(zeros1_h, dsh.at[pl.ds(start, SLICE)])
    pltpu.sync_copy(u_h, u_v)
    pltpu.sync_copy(v_h, v_v)
    pltpu.sync_copy(m_h, m_v)
    pltpu.sync_copy(b_h, b_v)
    issue_in(0, 0)
    issue_in(1, 1)
    plsc.subcore_barrier()
    Mv = m_v[...]
    Bv = b_v[...]

    def outer(i, carry):
        t0 = i * 4
        for b in range(4):
            t = t0 + b
            wait_in(b)

            @pl.when(t >= 2)
            def _(b=b):
                wait_out((b + 2) % 4, b % 2)

            @pl.when(t + 2 < CPT)
            def _(t=t, b=b):
                issue_in(t + 2, (b + 2) % 4)

            eb = ebufs[b]
            pb = pbufs[b % 2]

            def inner(j, icarry, eb=eb, pb=pb):
                sl = pl.ds(j * L, L)
                si = eb[0, sl]
                di = eb[1, sl]
                uu = plsc.load_gather(u_v, [si])
                vv = plsc.load_gather(v_v, [di])
                t0v = uu + vv + Bv
                e = jnp.where(t0v > 0, t0v, t0v * jnp.float32(0.01))
                p = jnp.exp(e - Mv)
                for k in range(K):
                    pb[k, sl] = p * eb[2 + k, sl].astype(jnp.float32)
                return icarry

            lax.fori_loop(0, CH // L, inner, 0)
            issue_out(b, b % 2)
        return carry

    lax.fori_loop(0, CPT // 4, outer, 0)
    wait_out(2, 0)
    wait_out(3, 1)
    plsc.subcore_barrier()
    for k, dsh in enumerate(dens):
        pltpu.sync_copy(dsh.at[pl.ds(start, SLICE)],
                        out_h.at[pl.ds((c * K + k) * NPAD + start, SLICE)])


@functools.partial(
    pl.kernel,
    out_type=jax.ShapeDtypeStruct((NC * K * NPAD,), jnp.float32),
    mesh=_mesh,
    compiler_params=_sc_params,
    scratch_types=[
        pltpu.VMEM((NPAD,), jnp.float32),
        pltpu.VMEM((NPAD,), jnp.float32),
        pltpu.VMEM((L,), jnp.float32),
        pltpu.VMEM((L,), jnp.float32),
        pltpu.VMEM((NPAD,), jnp.float32),
        pltpu.VMEM((NPAD,), jnp.float32),
        pltpu.VMEM((NPAD,), jnp.float32),
        pltpu.VMEM((NPAD,), jnp.float32),
        pltpu.VMEM((5, CH), jnp.int32),
        pltpu.VMEM((5, CH), jnp.int32),
        pltpu.VMEM((5, CH), jnp.int32),
        pltpu.VMEM((5, CH), jnp.int32),
        pltpu.VMEM((K, CH), jnp.float32),
        pltpu.VMEM((K, CH), jnp.float32),
        pltpu.SemaphoreType.DMA,
        pltpu.SemaphoreType.DMA,
        pltpu.SemaphoreType.DMA,
        pltpu.SemaphoreType.DMA,
        pltpu.SemaphoreType.DMA,
        pltpu.SemaphoreType.DMA,
        pltpu.MemorySpace.VMEM_SHARED((NPAD,), jnp.float32),
        pltpu.MemorySpace.VMEM_SHARED((NPAD,), jnp.float32),
        pltpu.MemorySpace.VMEM_SHARED((NPAD,), jnp.float32),
    ],
)
def _sc_w(u_h, v_h, m_h, b_h, e5_h, denparts_h, zeros1_h, out_h,
          u_v, v_v, m_v, b_v, dn0, dn1, dn2, tmp, e0, e1, e2b, e3, a0, a1,
          is0, is1, is2, is3, os0, os1, w0, w1, w2):
    c = lax.axis_index("c")
    s = lax.axis_index("s")
    wid = _wid()
    start = s * SLICE
    ebufs = (e0, e1, e2b, e3)
    abufs = (a0, a1)
    isems = (is0, is1, is2, is3)
    osems = (os0, os1)
    ws = (w0, w1, w2)
    dns = (dn0, dn1, dn2)

    def issue_in(t, b):
        pltpu.async_copy(e5_h.at[wid + NW * t], ebufs[b], isems[b])

    def wait_in(b):
        pltpu.make_async_copy(e5_h.at[0], ebufs[b], isems[b]).wait()

    def issue_out(b, p):
        for k in range(K):
            pltpu.async_copy(abufs[p].at[k], ws[k].at[ebufs[b].at[0]],
                             osems[p], add=True)

    def wait_out(b, p):
        for k in range(K):
            pltpu.make_async_copy(abufs[p].at[k],
                                  ws[k].at[ebufs[b].at[0]],
                                  osems[p]).wait()

    for wsh in ws:
        pltpu.sync_copy(zeros1_h, wsh.at[pl.ds(start, SLICE)])
    pltpu.sync_copy(u_h, u_v)
    pltpu.sync_copy(v_h, v_v)
    pltpu.sync_copy(m_h, m_v)
    pltpu.sync_copy(b_h, b_v)
    issue_in(0, 0)
    issue_in(1, 1)
    for k, dn in enumerate(dns):
        pltpu.sync_copy(denparts_h.at[pl.ds(k * NPAD, NPAD)], dn)
        pltpu.sync_copy(denparts_h.at[pl.ds((K + k) * NPAD, NPAD)], tmp)

        def dsum(j, carry, dn=dn):
            sl = pl.ds(j * L, L)
            dn[sl] = jnp.float32(1.0) / (dn[sl] + tmp[sl] + jnp.float32(1e-30))
            return carry

        lax.fori_loop(0, NPAD // L, dsum, 0)
    plsc.subcore_barrier()
    Mv = m_v[...]
    Bv = b_v[...]

    def outer(i, carry):
        t0 = i * 4
        for b in range(4):
            t = t0 + b
            wait_in(b)

            @pl.when(t >= 2)
            def _(b=b):
                wait_out((b + 2) % 4, b % 2)

            @pl.when(t + 2 < CPT)
            def _(t=t, b=b):
                issue_in(t + 2, (b + 2) % 4)

            eb = ebufs[b]
            ab = abufs[b % 2]

            def inner(j, icarry, eb=eb, ab=ab):
                sl = pl.ds(j * L, L)
                si = eb[0, sl]
                di = eb[1, sl]
                uu = plsc.load_gather(u_v, [si])
                vv = plsc.load_gather(v_v, [di])
                t0v = uu + vv + Bv
                e = jnp.where(t0v > 0, t0v, t0v * jnp.float32(0.01))
                p = jnp.exp(e - Mv)
                for k in range(K):
                    ddinv = plsc.load_gather(dns[k], [di])
                    mk = eb[2 + k, sl].astype(jnp.float32)
                    ab[k, sl] = p * mk * ddinv
                return icarry

            lax.fori_loop(0, CH // L, inner, 0)
            issue_out(b, b % 2)
        return carry

    lax.fori_loop(0, CPT // 4, outer, 0)
    wait_out(2, 0)
    wait_out(3, 1)
    plsc.subcore_barrier()
    for k, wsh in enumerate(ws):
        pltpu.sync_copy(wsh.at[pl.ds(start, SLICE)],
                        out_h.at[pl.ds((c * K + k) * NPAD + start, SLICE)])


def _tc7_body(wparts, z, zsum, wkw, wkb, lw4, lb, out, s_acc, a_acc):
    i = pl.program_id(0)
    wm = wparts[0] + wparts[1]
    sblk = lax.dot_general(wm, z[...], (((1,), (0,)), ((), ())),
                           preferred_element_type=jnp.float32)
    ablk = jnp.sum(wm, axis=1, keepdims=True)

    @pl.when(i == 0)
    def _():
        s_acc[...] = sblk
        a_acc[...] = ablk

    @pl.when(i > 0)
    def _():
        s_acc[...] = s_acc[...] + sblk
        a_acc[...] = a_acc[...] + ablk

    @pl.when(i == NBLK4 - 1)
    def _():
        acc = jnp.dot(zsum[...], lw4[0], preferred_element_type=jnp.float32)
        for k in range(K):
            pk = (jnp.dot(s_acc[k:k + 1, :], wkw[k],
                          preferred_element_type=jnp.float32)
                  + a_acc[k:k + 1, 0:1] * wkb[k:k + 1, :])
            acc = acc + jnp.dot(pk, lw4[k + 1],
                                preferred_element_type=jnp.float32)
        out[...] = acc / jnp.float32(N) + lb[...]


def _tc7(wparts, z, zsum, wkw, wkb, lw4, lb):
    return pl.pallas_call(
        _tc7_body,
        grid=(NBLK4,),
        in_specs=[
            pl.BlockSpec((NC, K, BLK4), lambda i: (0, 0, i)),
            pl.BlockSpec((BLK4, H), lambda i: (i, 0)),
            pl.BlockSpec((1, H), lambda i: (0, 0)),
            pl.BlockSpec((K, H, H), lambda i: (0, 0, 0)),
            pl.BlockSpec((K, H), lambda i: (0, 0)),
            pl.BlockSpec((K + 1, H, C), lambda i: (0, 0, 0)),
            pl.BlockSpec((1, C), lambda i: (0, 0)),
        ],
        out_specs=pl.BlockSpec((1, C), lambda i: (0, 0)),
        out_shape=jax.ShapeDtypeStruct((1, C), jnp.float32),
        scratch_shapes=[
            pltpu.VMEM((K, H), jnp.float32),
            pltpu.VMEM((K, 1), jnp.float32),
        ],
    )(wparts, z, zsum, wkw, wkb, lw4, lb)


def kernel(x, conv_w, conv_b, W_w, W_b, attn_w, attn_b, Wk_w, Wk_b,
           lin_w, lin_b, edge_index, motif_mask):
    pade = EP - E
    padi = (N + jnp.arange(pade, dtype=jnp.int32) % (NPAD - N)).astype(
        jnp.int32)
    srcdst = jnp.concatenate([edge_index, jnp.stack([padi, padi], 0)], 1)
    mmp = jnp.pad(motif_mask, ((0, 0), (0, pade)))
    e5 = jnp.concatenate([srcdst, mmp], 0).reshape(5, NCH2, CH).transpose(
        1, 0, 2)
    x_pad = jnp.pad(x, ((0, NPAD - N), (0, 0)))
    aw2 = jnp.concatenate([attn_w[:H], attn_w[H:]], axis=1)
    ones_ch = jnp.ones((CH,), jnp.float32)
    zeros1 = jnp.zeros((SLICE,), jnp.float32)
    zrows = jnp.zeros((SLICE, D), jnp.float32)

    degflat = _sc_deg(e5, ones_ch, zeros1)
    dcols = degflat.reshape(NC * 2, NPAD, 1)
    xn, ndcol = _tc2(dcols, x_pad)
    aggparts = _sc_agg(xn, e5, zrows)
    z, u2, v2, zsum, m16_2, b16_2 = _tc4(
        aggparts, xn, ndcol, conv_w, conv_b[None, :],
        W_w, W_b[None, :], aw2, attn_b[None, :])
    m16 = m16_2.reshape(L)
    b16 = b16_2.reshape(L)
    u = u2.reshape(NPAD)
    v = v2.reshape(NPAD)
    denflat = _sc_den(u, v, m16, b16, e5, zeros1)
    wflat = _sc_w(u, v, m16, b16, e5, denflat, zeros1)
    wparts = wflat.reshape(NC, K, NPAD)
    lw4 = lin_w.reshape(K + 1, H, C)
    return _tc7(wparts, z, zsum, Wk_w, Wk_b, lw4, lin_b[None, :])


# speedup vs baseline: 1.1182x; 1.1182x over previous
import functools

import jax
import jax.numpy as jnp
from jax import lax
from jax.experimental import pallas as pl
from jax.experimental.pallas import tpu as pltpu
from jax.experimental.pallas import tpu_sc as plsc

N = 10000
E = 320000
D = 128
H = 128
K = 3
C = 2

NC = 2
NS = 16
NW = NC * NS
L = 16
CH = 128
EP = 327680
NCH2 = EP // CH
CPT = NCH2 // NW
NPAD = 10240
SLICE = NPAD // NS
BLK = 256
NBLK = NPAD // BLK

_mesh = plsc.VectorSubcoreMesh(core_axis_name="c", subcore_axis_name="s")
_sc_params = pltpu.CompilerParams(needs_layout_passes=False)


def _wid():
    return lax.axis_index("s") * NC + lax.axis_index("c")


@functools.partial(
    pl.kernel,
    out_type=jax.ShapeDtypeStruct((NC * 2 * NPAD,), jnp.float32),
    mesh=_mesh,
    compiler_params=_sc_params,
    scratch_types=[
        pltpu.VMEM((CH,), jnp.float32),
        pltpu.VMEM((5, CH), jnp.int32),
        pltpu.VMEM((5, CH), jnp.int32),
        pltpu.VMEM((5, CH), jnp.int32),
        pltpu.VMEM((5, CH), jnp.int32),
        pltpu.SemaphoreType.DMA,
        pltpu.SemaphoreType.DMA,
        pltpu.SemaphoreType.DMA,
        pltpu.SemaphoreType.DMA,
        pltpu.SemaphoreType.DMA,
        pltpu.SemaphoreType.DMA,
        pltpu.MemorySpace.VMEM_SHARED((NPAD,), jnp.float32),
        pltpu.MemorySpace.VMEM_SHARED((NPAD,), jnp.float32),
    ],
)
def _sc_deg(e5_h, ones_h, zeros1_h, out_h, ones_v, e0, e1, e2b, e3,
            is0, is1, is2, is3, os0, os1, dego_sh, degi_sh):
    c = lax.axis_index("c")
    s = lax.axis_index("s")
    wid = _wid()
    start = s * SLICE
    ebufs = (e0, e1, e2b, e3)
    isems = (is0, is1, is2, is3)
    osems = (os0, os1)

    def issue_in(t, b):
        pltpu.async_copy(e5_h.at[wid + NW * t], ebufs[b], isems[b])

    def wait_in(b):
        pltpu.make_async_copy(e5_h.at[0], ebufs[b], isems[b]).wait()

    def issue_out(b, p):
        pltpu.async_copy(ones_v, dego_sh.at[ebufs[b].at[0]], osems[p],
                         add=True)
        pltpu.async_copy(ones_v, degi_sh.at[ebufs[b].at[1]], osems[p],
                         add=True)

    def wait_out(b, p):
        pltpu.make_async_copy(ones_v, dego_sh.at[ebufs[b].at[0]],
                              osems[p]).wait()
        pltpu.make_async_copy(ones_v, degi_sh.at[ebufs[b].at[1]],
                              osems[p]).wait()

    pltpu.sync_copy(zeros1_h, dego_sh.at[pl.ds(start, SLICE)])
    pltpu.sync_copy(zeros1_h, degi_sh.at[pl.ds(start, SLICE)])
    pltpu.sync_copy(ones_h, ones_v)
    issue_in(0, 0)
    issue_in(1, 1)
    plsc.subcore_barrier()

    def outer(i, carry):
        t0 = i * 4
        for b in range(4):
            t = t0 + b
            wait_in(b)

            @pl.when(t >= 2)
            def _(b=b):
                wait_out((b + 2) % 4, b % 2)

            @pl.when(t + 2 < CPT)
            def _(t=t, b=b):
                issue_in(t + 2, (b + 2) % 4)

            issue_out(b, b % 2)
        return carry

    lax.fori_loop(0, CPT // 4, outer, 0)
    wait_out(2, 0)
    wait_out(3, 1)
    plsc.subcore_barrier()
    pltpu.sync_copy(dego_sh.at[pl.ds(start, SLICE)],
                    out_h.at[pl.ds((c * 2 + 0) * NPAD + start, SLICE)])
    pltpu.sync_copy(degi_sh.at[pl.ds(start, SLICE)],
                    out_h.at[pl.ds((c * 2 + 1) * NPAD + start, SLICE)])


BLK2 = 512


def _ident(blk):
    return (lax.broadcasted_iota(jnp.int32, (blk, blk), 0)
            == lax.broadcasted_iota(jnp.int32, (blk, blk), 1)
            ).astype(jnp.float32)


_COLDN = (((1,), (1,)), ((), ()))


def _tc2_body(deg, x, xn_out, ident_s):
    i = pl.program_id(0)

    @pl.when(i == 0)
    def _():
        ident_s[...] = _ident(BLK2)

    d = deg[...]
    norm_s = lax.rsqrt(d[0:1] + d[2:3] + 1.0)
    ncol_s = lax.dot_general(ident_s[...], norm_s, _COLDN,
                             preferred_element_type=jnp.float32)
    xn_out[...] = x[...] * ncol_s


def _tc2(deg2d, x_pad):
    mat = pl.BlockSpec((BLK2, D), lambda i: (i, 0))
    return pl.pallas_call(
        _tc2_body,
        grid=(NPAD // BLK2,),
        in_specs=[pl.BlockSpec((4, BLK2), lambda i: (0, i)), mat],
        out_specs=mat,
        out_shape=jax.ShapeDtypeStruct((NPAD, D), jnp.float32),
        scratch_shapes=[pltpu.VMEM((BLK2, BLK2), jnp.float32)],
    )(deg2d, x_pad)


@functools.partial(
    pl.kernel,
    out_type=jax.ShapeDtypeStruct((NC, NPAD, D), jnp.float32),
    mesh=_mesh,
    compiler_params=_sc_params,
    scratch_types=(
        [pltpu.VMEM((5, CH), jnp.int32)] * 8 +
        [pltpu.VMEM((CH, D), jnp.float32)] * 2 +
        [pltpu.SemaphoreType.DMA] * 8 +
        [pltpu.SemaphoreType.DMA] * 2 +
        [pltpu.SemaphoreType.DMA] * 2 +
        [pltpu.MemorySpace.VMEM_SHARED((NPAD, D), jnp.float32)]
    ),
)
def _sc_agg(xn_h, e5_h, zrows_h, out_h, *refs):
    ebufs = refs[0:8]
    rows = refs[8:10]
    isems = refs[10:18]
    gsems = refs[18:20]
    ssems = refs[20:22]
    acc_sh = refs[22]
    c = lax.axis_index("c")
    s = lax.axis_index("s")
    wid = _wid()
    start = s * SLICE

    def issue_in(t, b):
        pltpu.async_copy(e5_h.at[wid + NW * t], ebufs[b], isems[b])

    def wait_in(b):
        pltpu.make_async_copy(e5_h.at[0], ebufs[b], isems[b]).wait()

    def issue_gather(b, p):
        pltpu.async_copy(xn_h.at[ebufs[b].at[0]], rows[p], gsems[p])

    def wait_gather(b, p):
        pltpu.make_async_copy(xn_h.at[ebufs[b].at[0]], rows[p],
                              gsems[p]).wait()

    def issue_scatter(b, p):
        pltpu.async_copy(rows[p], acc_sh.at[ebufs[b].at[1]], ssems[p],
                         add=True)

    def wait_scatter(b, p):
        pltpu.make_async_copy(rows[p], acc_sh.at[ebufs[b].at[1]],
                              ssems[p]).wait()

    pltpu.sync_copy(zrows_h, acc_sh.at[pl.ds(start, SLICE)])
    for b in range(5):
        issue_in(b, b)
    plsc.subcore_barrier()

    def outer(i, carry):
        t0 = i * 8
        for b in range(8):
            t = t0 + b
            wait_in(b)

            @pl.when(t >= 2)
            def _(b=b):
                wait_scatter((b + 6) % 8, b % 2)

            @pl.when(t + 5 < CPT)
            def _(t=t, b=b):
                issue_in(t + 5, (b + 5) % 8)

            issue_gather(b, b % 2)

            @pl.when(t >= 1)
            def _(b=b):
                wait_gather((b + 7) % 8, (b + 1) % 2)
                issue_scatter((b + 7) % 8, (b + 1) % 2)

        return carry

    lax.fori_loop(0, CPT // 8, outer, 0)
    wait_gather(7, 1)
    issue_scatter(7, 1)
    wait_scatter(6, 0)
    wait_scatter(7, 1)
    plsc.subcore_barrier()
    pltpu.sync_copy(acc_sh.at[pl.ds(start, SLICE)],
                    out_h.at[c, pl.ds(start, SLICE)])


BLK4 = 512
NBLK4 = NPAD // BLK4


def _tc4_body(aggparts, xn, deg, conv_w, conv_b, W_w, W_b, awT, ab,
              z_out, u_out, v_out, zsum_out, m16_out, b16_out,
              ident_s, muv_acc):
    i = pl.program_id(0)

    @pl.when(i == 0)
    def _():
        ident_s[...] = _ident(BLK4)

    d = deg[...]
    norm_d = lax.rsqrt(d[1:2] + d[3:4] + 1.0)
    ncol_d = lax.dot_general(ident_s[...], norm_d, _COLDN,
                             preferred_element_type=jnp.float32)
    A = (aggparts[0] + aggparts[1] + xn[...]) * ncol_d
    h = jnp.dot(A, conv_w[...], preferred_element_type=jnp.float32) + conv_b[...]
    Z = jnp.dot(h, W_w[...], preferred_element_type=jnp.float32) + W_b[...]
    UVr = lax.dot_general(awT[...], Z, (((1,), (1,)), ((), ())),
                          preferred_element_type=jnp.float32)
    z_out[...] = Z
    u_out[...] = UVr[0]
    v_out[...] = UVr[1]
    rows = lax.broadcasted_iota(jnp.int32, (BLK4, 1), 0) + i * BLK4
    valid = rows < N
    zs = jnp.sum(jnp.where(valid, Z, 0.0), axis=0, keepdims=True)
    lanes = lax.broadcasted_iota(jnp.int32, (1, BLK4), 1) + i * BLK4
    mu = jnp.max(jnp.where(lanes < N, UVr, -3e38), axis=1,
                 keepdims=True)

    @pl.when(i == 0)
    def _():
        zsum_out[...] = zs
        muv_acc[...] = mu

    @pl.when(i > 0)
    def _():
        zsum_out[...] = zsum_out[...] + zs
        muv_acc[...] = jnp.maximum(muv_acc[...], mu)

    @pl.when(i == NBLK4 - 1)
    def _():
        muv = muv_acc[...]
        m = jnp.maximum(muv[0:1, :] + muv[1:2, :] + ab[...], 0.0)
        m16_out[...] = jnp.broadcast_to(m, (1, L))
        b16_out[...] = jnp.broadcast_to(ab[...], (1, L))


def _tc4(aggparts, xn, deg2d, conv_w, conv_b, W_w, W_b, awT, ab):
    return pl.pallas_call(
        _tc4_body,
        grid=(NBLK4,),
        in_specs=[
            pl.BlockSpec((NC, BLK4, D), lambda i: (0, i, 0)),
            pl.BlockSpec((BLK4, D), lambda i: (i, 0)),
            pl.BlockSpec((4, BLK4), lambda i: (0, i)),
            pl.BlockSpec((D, D), lambda i: (0, 0)),
            pl.BlockSpec((1, D), lambda i: (0, 0)),
            pl.BlockSpec((D, H), lambda i: (0, 0)),
            pl.BlockSpec((1, H), lambda i: (0, 0)),
            pl.BlockSpec((2, H), lambda i: (0, 0)),
            pl.BlockSpec((1, 1), lambda i: (0, 0)),
        ],
        out_specs=[
            pl.BlockSpec((BLK4, H), lambda i: (i, 0)),
            pl.BlockSpec((BLK4,), lambda i: (i,)),
            pl.BlockSpec((BLK4,), lambda i: (i,)),
            pl.BlockSpec((1, H), lambda i: (0, 0)),
            pl.BlockSpec((1, L), lambda i: (0, 0)),
            pl.BlockSpec((1, L), lambda i: (0, 0)),
        ],
        out_shape=[
            jax.ShapeDtypeStruct((NPAD, H), jnp.float32),
            jax.ShapeDtypeStruct((NPAD,), jnp.float32),
            jax.ShapeDtypeStruct((NPAD,), jnp.float32),
            jax.ShapeDtypeStruct((1, H), jnp.float32),
            jax.ShapeDtypeStruct((1, L), jnp.float32),
            jax.ShapeDtypeStruct((1, L), jnp.float32),
        ],
        scratch_shapes=[
            pltpu.VMEM((BLK4, BLK4), jnp.float32),
            pltpu.VMEM((2, 1), jnp.float32),
        ],
    )(aggparts, xn, deg2d, conv_w, conv_b, W_w, W_b, awT, ab)


@functools.partial(
    pl.kernel,
    out_type=jax.ShapeDtypeStruct((NC * K * NPAD,), jnp.float32),
    mesh=_mesh,
    compiler_params=_sc_params,
    scratch_types=[
        pltpu.VMEM((NPAD,), jnp.float32),
        pltpu.VMEM((NPAD,), jnp.float32),
        pltpu.VMEM((L,), jnp.float32),
        pltpu.VMEM((L,), jnp.float32),
        pltpu.VMEM((5, CH), jnp.int32),
        pltpu.VMEM((5, CH), jnp.int32),
        pltpu.VMEM((5, CH), jnp.int32),
        pltpu.VMEM((5, CH), jnp.int32),
        pltpu.VMEM((K, CH), jnp.float32),
        pltpu.VMEM((K, CH), jnp.float32),
        pltpu.SemaphoreType.DMA,
        pltpu.SemaphoreType.DMA,
        pltpu.SemaphoreType.DMA,
        pltpu.SemaphoreType.DMA,
        pltpu.SemaphoreType.DMA,
        pltpu.SemaphoreType.DMA,
        pltpu.MemorySpace.VMEM_SHARED((NPAD,), jnp.float32),
        pltpu.MemorySpace.VMEM_SHARED((NPAD,), jnp.float32),
        pltpu.MemorySpace.VMEM_SHARED((NPAD,), jnp.float32),
    ],
)
def _sc_den(u_h, v_h, m_h, b_h, e5_h, zeros1_h, out_h,
            u_v, v_v, m_v, b_v, e0, e1, e2b, e3, p0, p1,
            is0, is1, is2, is3, os0, os1, d0, d1, d2):
    c = lax.axis_index("c")
    s = lax.axis_index("s")
    wid = _wid()
    start = s * SLICE
    ebufs = (e0, e1, e2b, e3)
    pbufs = (p0, p1)
    isems = (is0, is1, is2, is3)
    osems = (os0, os1)
    dens = (d0, d1, d2)

    def issue_in(t, b):
        pltpu.async_copy(e5_h.at[wid + NW * t], ebufs[b], isems[b])

    def wait_in(b):
        pltpu.make_async_copy(e5_h.at[0], ebufs[b], isems[b]).wait()

    def issue_out(b, p):
        for k in range(K):
            pltpu.async_copy(pbufs[p].at[k], dens[k].at[ebufs[b].at[1]],
                             osems[p], add=True)

    def wait_out(b, p):
        for k in range(K):
            pltpu.make_async_copy(pbufs[p].at[k],
                                  dens[k].at[ebufs[b].at[1]],
                                  osems[p]).wait()

    for dsh in dens:
        pltpu.sync_copy(zeros1_h, dsh.at[pl.ds(start, SLICE)])
    pltpu.sync_copy(u_h, u_v)
    pltpu.sync_copy(v_h, v_v)
    pltpu.sync_copy(m_h, m_v)
    pltpu.sync_copy(b_h, b_v)
    issue_in(0, 0)
    issue_in(1, 1)
    plsc.subcore_barrier()
    Mv = m_v[...]
    Bv = b_v[...]

    def outer(i, carry):
        t0 = i * 4
        for b in range(4):
            t = t0 + b
            wait_in(b)

            @pl.when(t >= 2)
            def _(b=b):
                wait_out((b + 2) % 4, b % 2)

            @pl.when(t + 2 < CPT)
            def _(t=t, b=b):
                issue_in(t + 2, (b + 2) % 4)

            eb = ebufs[b]
            pb = pbufs[b % 2]

            def inner(j, icarry, eb=eb, pb=pb):
                sl = pl.ds(j * L, L)
                si = eb[0, sl]
                di = eb[1, sl]
                uu = plsc.load_gather(u_v, [si])
                vv = plsc.load_gather(v_v, [di])
                t0v = uu + vv + Bv
                e = jnp.where(t0v > 0, t0v, t0v * jnp.float32(0.01))
                p = jnp.exp(e - Mv)
                for k in range(K):
                    pb[k, sl] = p * eb[2 + k, sl].astype(jnp.float32)
                return icarry

            lax.fori_loop(0, CH // L, inner, 0)
            issue_out(b, b % 2)
        return carry

    lax.fori_loop(0, CPT // 4, outer, 0)
    wait_out(2, 0)
    wait_out(3, 1)
    plsc.subcore_barrier()
    for k, dsh in enumerate(dens):
        pltpu.sync_copy(dsh.at[pl.ds(start, SLICE)],
                        out_h.at[pl.ds((c * K + k) * NPAD + start, SLICE)])


@functools.partial(
    pl.kernel,
    out_type=jax.ShapeDtypeStruct((NC * K * NPAD,), jnp.float32),
    mesh=_mesh,
    compiler_params=_sc_params,
    scratch_types=[
        pltpu.VMEM((NPAD,), jnp.float32),
        pltpu.VMEM((NPAD,), jnp.float32),
        pltpu.VMEM((L,), jnp.float32),
        pltpu.VMEM((L,), jnp.float32),
        pltpu.VMEM((NPAD,), jnp.float32),
        pltpu.VMEM((NPAD,), jnp.float32),
        pltpu.VMEM((NPAD,), jnp.float32),
        pltpu.VMEM((NPAD,), jnp.float32),
        pltpu.VMEM((5, CH), jnp.int32),
        pltpu.VMEM((5, CH), jnp.int32),
        pltpu.VMEM((5, CH), jnp.int32),
        pltpu.VMEM((5, CH), jnp.int32),
        pltpu.VMEM((K, CH), jnp.float32),
        pltpu.VMEM((K, CH), jnp.float32),
        pltpu.SemaphoreType.DMA,
        pltpu.SemaphoreType.DMA,
        pltpu.SemaphoreType.DMA,
        pltpu.SemaphoreType.DMA,
        pltpu.SemaphoreType.DMA,
        pltpu.SemaphoreType.DMA,
        pltpu.MemorySpace.VMEM_SHARED((NPAD,), jnp.float32),
        pltpu.MemorySpace.VMEM_SHARED((NPAD,), jnp.float32),
        pltpu.MemorySpace.VMEM_SHARED((NPAD,), jnp.float32),
    ],
)
def _sc_w(u_h, v_h, m_h, b_h, e5_h, denparts_h, zeros1_h, out_h,
          u_v, v_v, m_v, b_v, dn0, dn1, dn2, tmp, e0, e1, e2b, e3, a0, a1,
          is0, is1, is2, is3, os0, os1, w0, w1, w2):
    c = lax.axis_index("c")
    s = lax.axis_index("s")
    wid = _wid()
    start = s * SLICE
    ebufs = (e0, e1, e2b, e3)
    abufs = (a0, a1)
    isems = (is0, is1, is2, is3)
    osems = (os0, os1)
    ws = (w0, w1, w2)
    dns = (dn0, dn1, dn2)

    def issue_in(t, b):
        pltpu.async_copy(e5_h.at[wid + NW * t], ebufs[b], isems[b])

    def wait_in(b):
        pltpu.make_async_copy(e5_h.at[0], ebufs[b], isems[b]).wait()

    def issue_out(b, p):
        for k in range(K):
            pltpu.async_copy(abufs[p].at[k], ws[k].at[ebufs[b].at[0]],
                             osems[p], add=True)

    def wait_out(b, p):
        for k in range(K):
            pltpu.make_async_copy(abufs[p].at[k],
                                  ws[k].at[ebufs[b].at[0]],
                                  osems[p]).wait()

    for wsh in ws:
        pltpu.sync_copy(zeros1_h, wsh.at[pl.ds(start, SLICE)])
    pltpu.sync_copy(u_h, u_v)
    pltpu.sync_copy(v_h, v_v)
    pltpu.sync_copy(m_h, m_v)
    pltpu.sync_copy(b_h, b_v)
    issue_in(0, 0)
    issue_in(1, 1)
    for k, dn in enumerate(dns):
        pltpu.sync_copy(denparts_h.at[pl.ds(k * NPAD, NPAD)], dn)
        pltpu.sync_copy(denparts_h.at[pl.ds((K + k) * NPAD, NPAD)], tmp)

        def dsum(j, carry, dn=dn):
            sl = pl.ds(j * L, L)
            dn[sl] = dn[sl] + tmp[sl]
            return carry

        lax.fori_loop(0, NPAD // L, dsum, 0)
    plsc.subcore_barrier()
    Mv = m_v[...]
    Bv = b_v[...]

    def outer(i, carry):
        t0 = i * 4
        for b in range(4):
            t = t0 + b
            wait_in(b)

            @pl.when(t >= 2)
            def _(b=b):
                wait_out((b + 2) % 4, b % 2)

            @pl.when(t + 2 < CPT)
            def _(t=t, b=b):
                issue_in(t + 2, (b + 2) % 4)

            eb = ebufs[b]
            ab = abufs[b % 2]

            def inner(j, icarry, eb=eb, ab=ab):
                sl = pl.ds(j * L, L)
                si = eb[0, sl]
                di = eb[1, sl]
                uu = plsc.load_gather(u_v, [si])
                vv = plsc.load_gather(v_v, [di])
                t0v = uu + vv + Bv
                e = jnp.where(t0v > 0, t0v, t0v * jnp.float32(0.01))
                p = jnp.exp(e - Mv)
                for k in range(K):
                    dd = plsc.load_gather(dns[k], [di])
                    mk = eb[2 + k, sl].astype(jnp.float32)
                    ab[k, sl] = (p * mk) / (dd + jnp.float32(1e-30))
                return icarry

            lax.fori_loop(0, CH // L, inner, 0)
            issue_out(b, b % 2)
        return carry

    lax.fori_loop(0, CPT // 4, outer, 0)
    wait_out(2, 0)
    wait_out(3, 1)
    plsc.subcore_barrier()
    for k, wsh in enumerate(ws):
        pltpu.sync_copy(wsh.at[pl.ds(start, SLICE)],
                        out_h.at[pl.ds((c * K + k) * NPAD + start, SLICE)])


def _tc7_body(wparts, z, zsum, wkw, wkb, lw4, lb, out, s_acc, a_acc):
    i = pl.program_id(0)
    wm = wparts[0] + wparts[1]
    sblk = lax.dot_general(wm, z[...], (((1,), (0,)), ((), ())),
                           preferred_element_type=jnp.float32)
    ablk = jnp.sum(wm, axis=1, keepdims=True)

    @pl.when(i == 0)
    def _():
        s_acc[...] = sblk
        a_acc[...] = ablk

    @pl.when(i > 0)
    def _():
        s_acc[...] = s_acc[...] + sblk
        a_acc[...] = a_acc[...] + ablk

    @pl.when(i == NBLK4 - 1)
    def _():
        acc = jnp.dot(zsum[...], lw4[0], preferred_element_type=jnp.float32)
        for k in range(K):
            pk = (jnp.dot(s_acc[k:k + 1, :], wkw[k],
                          preferred_element_type=jnp.float32)
                  + a_acc[k:k + 1, 0:1] * wkb[k:k + 1, :])
            acc = acc + jnp.dot(pk, lw4[k + 1],
                                preferred_element_type=jnp.float32)
        out[...] = acc / jnp.float32(N) + lb[...]


def _tc7(wparts, z, zsum, wkw, wkb, lw4, lb):
    return pl.pallas_call(
        _tc7_body,
        grid=(NBLK4,),
        in_specs=[
            pl.BlockSpec((NC, K, BLK4), lambda i: (0, 0, i)),
            pl.BlockSpec((BLK4, H), lambda i: (i, 0)),
            pl.BlockSpec((1, H), lambda i: (0, 0)),
            pl.BlockSpec((K, H, H), lambda i: (0, 0, 0)),
            pl.BlockSpec((K, H), lambda i: (0, 0)),
            pl.BlockSpec((K + 1, H, C), lambda i: (0, 0, 0)),
            pl.BlockSpec((1, C), lambda i: (0, 0)),
        ],
        out_specs=pl.BlockSpec((1, C), lambda i: (0, 0)),
        out_shape=jax.ShapeDtypeStruct((1, C), jnp.float32),
        scratch_shapes=[
            pltpu.VMEM((K, H), jnp.float32),
            pltpu.VMEM((K, 1), jnp.float32),
        ],
    )(wparts, z, zsum, wkw, wkb, lw4, lb)


def kernel(x, conv_w, conv_b, W_w, W_b, attn_w, attn_b, Wk_w, Wk_b,
           lin_w, lin_b, edge_index, motif_mask):
    pade = EP - E
    padi = (N + jnp.arange(pade, dtype=jnp.int32) % (NPAD - N)).astype(
        jnp.int32)
    srcdst = jnp.concatenate([edge_index, jnp.stack([padi, padi], 0)], 1)
    mmp = jnp.pad(motif_mask, ((0, 0), (0, pade)))
    e5 = jnp.concatenate([srcdst, mmp], 0).reshape(5, NCH2, CH).transpose(
        1, 0, 2)
    x_pad = jnp.pad(x, ((0, NPAD - N), (0, 0)))
    awT = attn_w[:, 0].reshape(2, H)
    ones_ch = jnp.ones((CH,), jnp.float32)
    zeros1 = jnp.zeros((SLICE,), jnp.float32)
    zrows = jnp.zeros((SLICE, D), jnp.float32)

    degflat = _sc_deg(e5, ones_ch, zeros1)
    deg2d = degflat.reshape(NC * 2, NPAD)
    xn = _tc2(deg2d, x_pad)
    aggparts = _sc_agg(xn, e5, zrows)
    z, u, v, zsum, m16_2, b16_2 = _tc4(
        aggparts, xn, deg2d, conv_w, conv_b[None, :],
        W_w, W_b[None, :], awT, attn_b[None, :])
    m16 = m16_2.reshape(L)
    b16 = b16_2.reshape(L)
    denflat = _sc_den(u, v, m16, b16, e5, zeros1)
    wflat = _sc_w(u, v, m16, b16, e5, denflat, zeros1)
    wparts = wflat.reshape(NC, K, NPAD)
    lw4 = lin_w.reshape(K + 1, H, C)
    return _tc7(wparts, z, zsum, Wk_w, Wk_b, lw4, lin_b[None, :])
